# initial kernel scaffold (unmeasured)
import jax
import jax.numpy as jnp
from jax import lax
from jax.experimental import pallas as pl
from jax.experimental.pallas import tpu as pltpu

N_DEV = 8
SQ = 2048
D = 1024
HQ = 8
DH = 128
WIN = 128
CHUNK = 256
N_CHUNK = SQ // CHUNK
BAND = 512
EDGE = 256
KTOT = SQ + EDGE
SCALE = 0.08838834764831843


def kernel(x, Wq, K_ext, V_ext, Wo):
    def body(x_ref, wq_ref, k_ref, v_ref, wo_ref, out_ref,
             kall, vall, stage, estage, edge,
             lsem, esend, erecv, ssend, srecv):
        my = lax.axis_index("i")

        def edge_rdma(dev):
            return pltpu.make_async_remote_copy(
                src_ref=edge, dst_ref=edge,
                send_sem=esend, recv_sem=erecv,
                device_id=(dev,), device_id_type=pl.DeviceIdType.MESH,
            )

        def chunk_rdma(c, dev):
            sl = (0, pl.ds(c * CHUNK, CHUNK), slice(None))
            return pltpu.make_async_remote_copy(
                src_ref=out_ref.at[sl], dst_ref=out_ref.at[sl],
                send_sem=ssend.at[c], recv_sem=srecv.at[c],
                device_id=(dev,), device_id_type=pl.DeviceIdType.MESH,
            )

        @pl.when(my == 1)
        def _():
            cpk = pltpu.make_async_copy(
                k_ref.at[0, pl.ds(0, EDGE)], estage.at[0], lsem.at[0])
            cpv = pltpu.make_async_copy(
                v_ref.at[0, pl.ds(0, EDGE)], estage.at[1], lsem.at[1])
            cpk.start()
            cpv.start()
            cpk.wait()
            cpv.wait()
            edge[0] = estage[0].reshape(EDGE, D).astype(jnp.bfloat16)
            edge[1] = estage[1].reshape(EDGE, D).astype(jnp.bfloat16)
            snd = edge_rdma(0)
            snd.start()
            snd.wait_send()

        @pl.when(my == 0)
        def _():
            cpk = pltpu.make_async_copy(k_ref.at[0], stage, lsem.at[0])
            cpk.start()
            cpk.wait()
            kall[pl.ds(0, SQ), :] = stage[...].reshape(SQ, D).astype(jnp.bfloat16)
            cpv = pltpu.make_async_copy(v_ref.at[0], stage, lsem.at[1])
            cpv.start()
            cpv.wait()
            vall[pl.ds(0, SQ), :] = stage[...].reshape(SQ, D).astype(jnp.bfloat16)

            rcv = edge_rdma(1)
            rcv.wait_recv()
            kall[pl.ds(SQ, EDGE), :] = edge[0]
            vall[pl.ds(SQ, EDGE), :] = edge[1]

            wq = wq_ref[...].astype(jnp.bfloat16)
            wo = wo_ref[...].astype(jnp.bfloat16)
            sends = []
            for c in range(N_CHUNK):
                o = max(0, c * CHUNK - WIN)
                xc = x_ref[0, pl.ds(c * CHUNK, CHUNK), :].astype(jnp.bfloat16)
                q = jnp.dot(xc, wq, preferred_element_type=jnp.float32)
                q = (q * SCALE).astype(jnp.bfloat16)
                kb = kall[pl.ds(o, BAND), :]
                vb = vall[pl.ds(o, BAND), :]
                qi = c * CHUNK + lax.broadcasted_iota(jnp.int32, (CHUNK, BAND), 0)
                ki = o + lax.broadcasted_iota(jnp.int32, (CHUNK, BAND), 1)
                neg = jnp.where(jnp.abs(qi - ki) <= WIN, 0.0, -1e9).astype(jnp.float32)
                ctx_cols = []
                for h in range(HQ):
                    qh = q[:, h * DH:(h + 1) * DH]
                    kh = kb[:, h * DH:(h + 1) * DH]
                    s = lax.dot_general(
                        qh, kh, (((1,), (1,)), ((), ())),
                        preferred_element_type=jnp.float32,
                    ) + neg
                    m = jnp.max(s, axis=1, keepdims=True)
                    e = jnp.exp(s - m)
                    w = (e / jnp.sum(e, axis=1, keepdims=True)).astype(jnp.bfloat16)
                    vh = vb[:, h * DH:(h + 1) * DH]
                    ctx_cols.append(
                        jnp.dot(w, vh, preferred_element_type=jnp.float32
                                ).astype(jnp.bfloat16))
                ctx = jnp.concatenate(ctx_cols, axis=1)
                outc = jnp.dot(ctx, wo, preferred_element_type=jnp.float32)
                out_ref[0, pl.ds(c * CHUNK, CHUNK), :] = outc.astype(jnp.bfloat16)
                snd = chunk_rdma(c, 1)
                snd.start()
                sends.append(snd)
            for snd in sends:
                snd.wait_send()

        @pl.when(my > 0)
        def _():
            for c in range(N_CHUNK):
                chunk_rdma(c, 0).wait_recv()

                @pl.when(my < N_DEV - 1)
                def _():
                    fwd = chunk_rdma(c, my + 1)
                    fwd.start()
                    fwd.wait_send()

        return

    return pl.pallas_call(
        body,
        out_shape=jax.ShapeDtypeStruct((1, SQ, D), jnp.bfloat16),
        in_specs=[
            pl.BlockSpec(memory_space=pltpu.VMEM),
            pl.BlockSpec(memory_space=pltpu.VMEM),
            pl.BlockSpec(memory_space=pltpu.ANY),
            pl.BlockSpec(memory_space=pltpu.ANY),
            pl.BlockSpec(memory_space=pltpu.VMEM),
        ],
        out_specs=pl.BlockSpec(memory_space=pltpu.VMEM),
        scratch_shapes=[
            pltpu.VMEM((KTOT, D), jnp.bfloat16),
            pltpu.VMEM((KTOT, D), jnp.bfloat16),
            pltpu.VMEM((SQ, HQ, DH), jnp.float32),
            pltpu.VMEM((2, EDGE, HQ, DH), jnp.float32),
            pltpu.VMEM((2, EDGE, D), jnp.bfloat16),
            pltpu.SemaphoreType.DMA((2,)),
            pltpu.SemaphoreType.DMA,
            pltpu.SemaphoreType.DMA,
            pltpu.SemaphoreType.DMA((N_CHUNK,)),
            pltpu.SemaphoreType.DMA((N_CHUNK,)),
        ],
    )(x, Wq, K_ext, V_ext, Wo)


# baseline (device time: 105695 ns/iter reference)
import jax
import jax.numpy as jnp
from jax import lax
from jax.experimental import pallas as pl
from jax.experimental.pallas import tpu as pltpu

N_DEV = 8
SQ = 2048
D = 1024
HQ = 8
DH = 128
WIN = 128
CHUNK = 256
N_CHUNK = SQ // CHUNK
BAND = 512
EDGE = 256
KTOT = SQ + EDGE
SCALE = 0.08838834764831843


def kernel(x, Wq, K_ext, V_ext, Wo):
    def body(x_ref, wq_ref, k_ref, v_ref, wo_ref, out_ref,
             kall, vall, stage, estage, edge,
             lsem, esend, erecv, ssend, srecv):
        my = lax.axis_index("i")

        def edge_rdma(dev):
            return pltpu.make_async_remote_copy(
                src_ref=edge, dst_ref=edge,
                send_sem=esend, recv_sem=erecv,
                device_id=(dev,), device_id_type=pl.DeviceIdType.MESH,
            )

        def chunk_rdma(c, dev):
            sl = (0, pl.ds(c * CHUNK, CHUNK), slice(None))
            return pltpu.make_async_remote_copy(
                src_ref=out_ref.at[sl], dst_ref=out_ref.at[sl],
                send_sem=ssend.at[c], recv_sem=srecv.at[c],
                device_id=(dev,), device_id_type=pl.DeviceIdType.MESH,
            )

        @pl.when(my == 1)
        def _():
            cpk = pltpu.make_async_copy(
                k_ref.at[0, pl.ds(0, EDGE)], estage.at[0], lsem.at[0])
            cpv = pltpu.make_async_copy(
                v_ref.at[0, pl.ds(0, EDGE)], estage.at[1], lsem.at[1])
            cpk.start()
            cpv.start()
            cpk.wait()
            cpv.wait()
            edge[0] = estage[0].reshape(EDGE, D).astype(jnp.bfloat16)
            edge[1] = estage[1].reshape(EDGE, D).astype(jnp.bfloat16)
            snd = edge_rdma(0)
            snd.start()
            snd.wait_send()

        @pl.when(my == 0)
        def _():
            cpk = pltpu.make_async_copy(k_ref.at[0], stage, lsem.at[0])
            cpk.start()
            cpk.wait()
            kall[pl.ds(0, SQ), :] = stage[...].reshape(SQ, D).astype(jnp.bfloat16)
            cpv = pltpu.make_async_copy(v_ref.at[0], stage, lsem.at[1])
            cpv.start()
            cpv.wait()
            vall[pl.ds(0, SQ), :] = stage[...].reshape(SQ, D).astype(jnp.bfloat16)

            rcv = edge_rdma(1)
            rcv.wait_recv()
            kall[pl.ds(SQ, EDGE), :] = edge[0]
            vall[pl.ds(SQ, EDGE), :] = edge[1]

            wq = wq_ref[...].astype(jnp.bfloat16)
            wo = wo_ref[...].astype(jnp.bfloat16)
            sends = []
            for c in range(N_CHUNK):
                o = max(0, c * CHUNK - WIN)
                xc = x_ref[0, pl.ds(c * CHUNK, CHUNK), :].astype(jnp.bfloat16)
                q = jnp.dot(xc, wq, preferred_element_type=jnp.float32)
                q = (q * SCALE).astype(jnp.bfloat16)
                kb = kall[pl.ds(o, BAND), :]
                vb = vall[pl.ds(o, BAND), :]
                qi = c * CHUNK + lax.broadcasted_iota(jnp.int32, (CHUNK, BAND), 0)
                ki = o + lax.broadcasted_iota(jnp.int32, (CHUNK, BAND), 1)
                neg = jnp.where(jnp.abs(qi - ki) <= WIN, 0.0, -1e9).astype(jnp.float32)
                ctx_cols = []
                for h in range(HQ):
                    qh = q[:, h * DH:(h + 1) * DH]
                    kh = kb[:, h * DH:(h + 1) * DH]
                    s = lax.dot_general(
                        qh, kh, (((1,), (1,)), ((), ())),
                        preferred_element_type=jnp.float32,
                    ) + neg
                    m = jnp.max(s, axis=1, keepdims=True)
                    e = jnp.exp(s - m)
                    w = (e / jnp.sum(e, axis=1, keepdims=True)).astype(jnp.bfloat16)
                    vh = vb[:, h * DH:(h + 1) * DH]
                    ctx_cols.append(
                        jnp.dot(w, vh, preferred_element_type=jnp.float32
                                ).astype(jnp.bfloat16))
                ctx = jnp.concatenate(ctx_cols, axis=1)
                outc = jnp.dot(ctx, wo, preferred_element_type=jnp.float32)
                out_ref[0, pl.ds(c * CHUNK, CHUNK), :] = outc.astype(jnp.bfloat16)
                snd = chunk_rdma(c, 1)
                snd.start()
                sends.append(snd)
            for snd in sends:
                snd.wait_send()

        @pl.when(my > 0)
        def _():
            for c in range(N_CHUNK):
                chunk_rdma(c, 0).wait_recv()

                @pl.when(my < N_DEV - 1)
                def _():
                    fwd = chunk_rdma(c, my + 1)
                    fwd.start()
                    fwd.wait_send()

        return

    return pl.pallas_call(
        body,
        out_shape=jax.ShapeDtypeStruct((1, SQ, D), jnp.bfloat16),
        in_specs=[
            pl.BlockSpec(memory_space=pltpu.VMEM),
            pl.BlockSpec(memory_space=pltpu.VMEM),
            pl.BlockSpec(memory_space=pltpu.MemorySpace.HBM),
            pl.BlockSpec(memory_space=pltpu.MemorySpace.HBM),
            pl.BlockSpec(memory_space=pltpu.VMEM),
        ],
        out_specs=pl.BlockSpec(memory_space=pltpu.VMEM),
        scratch_shapes=[
            pltpu.VMEM((KTOT, D), jnp.bfloat16),
            pltpu.VMEM((KTOT, D), jnp.bfloat16),
            pltpu.VMEM((SQ, HQ, DH), jnp.float32),
            pltpu.VMEM((2, EDGE, HQ, DH), jnp.float32),
            pltpu.VMEM((2, EDGE, D), jnp.bfloat16),
            pltpu.SemaphoreType.DMA((2,)),
            pltpu.SemaphoreType.DMA,
            pltpu.SemaphoreType.DMA,
            pltpu.SemaphoreType.DMA((N_CHUNK,)),
            pltpu.SemaphoreType.DMA((N_CHUNK,)),
        ],
    )(x, Wq, K_ext, V_ext, Wo)


# device time: 81627 ns/iter; 1.2949x vs baseline; 1.2949x over previous
import jax
import jax.numpy as jnp
from jax import lax
from jax.experimental import pallas as pl
from jax.experimental.pallas import tpu as pltpu

N_DEV = 8
SQ = 2048
D = 1024
HQ = 8
DH = 128
WIN = 128
CHUNK = 256
N_CHUNK = SQ // CHUNK
BAND = 512
EDGE = 256
KTOT = SQ + EDGE
SCALE = 0.08838834764831843

TREE_CHILDREN = {0: (4, 3, 1), 4: (7, 5), 3: (2,), 7: (6,)}
MAX_FANOUT = 3


def kernel(x, Wq, K_ext, V_ext, Wo):
    def body(x_ref, wq_ref, k_ref, v_ref, wo_ref, out_ref,
             kall, vall, stage, estage, edge,
             lsem, esend, erecv, ssend, srecv):
        my = lax.axis_index("i")

        def edge_rdma(dev):
            return pltpu.make_async_remote_copy(
                src_ref=edge, dst_ref=edge,
                send_sem=esend, recv_sem=erecv,
                device_id=(dev,), device_id_type=pl.DeviceIdType.MESH,
            )

        def chunk_rdma(c, j, dev):
            sl = (0, pl.ds(c * CHUNK, CHUNK), slice(None))
            return pltpu.make_async_remote_copy(
                src_ref=out_ref.at[sl], dst_ref=out_ref.at[sl],
                send_sem=ssend.at[c, j], recv_sem=srecv.at[c],
                device_id=(dev,), device_id_type=pl.DeviceIdType.MESH,
            )

        @pl.when(my == 1)
        def _():
            cpk = pltpu.make_async_copy(
                k_ref.at[0, pl.ds(0, EDGE)], estage.at[0], lsem.at[0])
            cpv = pltpu.make_async_copy(
                v_ref.at[0, pl.ds(0, EDGE)], estage.at[1], lsem.at[1])
            cpk.start()
            cpv.start()
            cpk.wait()
            cpv.wait()
            edge[0] = estage[0].reshape(EDGE, D).astype(jnp.bfloat16)
            edge[1] = estage[1].reshape(EDGE, D).astype(jnp.bfloat16)
            snd = edge_rdma(0)
            snd.start()
            snd.wait_send()

        @pl.when(my == 0)
        def _():
            cpk = pltpu.make_async_copy(k_ref.at[0], stage.at[0], lsem.at[0])
            cpv = pltpu.make_async_copy(v_ref.at[0], stage.at[1], lsem.at[1])
            cpk.start()
            cpv.start()
            cpk.wait()
            kall[pl.ds(0, SQ), :] = stage[0].reshape(SQ, D).astype(jnp.bfloat16)
            cpv.wait()
            vall[pl.ds(0, SQ), :] = stage[1].reshape(SQ, D).astype(jnp.bfloat16)

            wq = wq_ref[...].astype(jnp.bfloat16)
            wo = wo_ref[...].astype(jnp.bfloat16)
            sends = []
            for c in range(N_CHUNK):
                o = max(0, c * CHUNK - WIN)
                if o + BAND > SQ:
                    rcv = edge_rdma(1)
                    rcv.wait_recv()
                    kall[pl.ds(SQ, EDGE), :] = edge[0]
                    vall[pl.ds(SQ, EDGE), :] = edge[1]
                xc = x_ref[0, pl.ds(c * CHUNK, CHUNK), :].astype(jnp.bfloat16)
                q = jnp.dot(xc, wq, preferred_element_type=jnp.float32)
                q = (q * SCALE).astype(jnp.bfloat16)
                kb = kall[pl.ds(o, BAND), :]
                vb = vall[pl.ds(o, BAND), :]
                qi = c * CHUNK + lax.broadcasted_iota(jnp.int32, (CHUNK, BAND), 0)
                ki = o + lax.broadcasted_iota(jnp.int32, (CHUNK, BAND), 1)
                neg = jnp.where(jnp.abs(qi - ki) <= WIN, 0.0, -1e9).astype(jnp.float32)
                ctx_cols = []
                for h in range(HQ):
                    qh = q[:, h * DH:(h + 1) * DH]
                    kh = kb[:, h * DH:(h + 1) * DH]
                    s = lax.dot_general(
                        qh, kh, (((1,), (1,)), ((), ())),
                        preferred_element_type=jnp.float32,
                    ) + neg
                    m = jnp.max(s, axis=1, keepdims=True)
                    e = jnp.exp(s - m)
                    w = (e / jnp.sum(e, axis=1, keepdims=True)).astype(jnp.bfloat16)
                    vh = vb[:, h * DH:(h + 1) * DH]
                    ctx_cols.append(
                        jnp.dot(w, vh, preferred_element_type=jnp.float32
                                ).astype(jnp.bfloat16))
                ctx = jnp.concatenate(ctx_cols, axis=1)
                outc = jnp.dot(ctx, wo, preferred_element_type=jnp.float32)
                out_ref[0, pl.ds(c * CHUNK, CHUNK), :] = outc.astype(jnp.bfloat16)
                for j, child in enumerate(TREE_CHILDREN[0]):
                    snd = chunk_rdma(c, j, child)
                    snd.start()
                    sends.append(snd)
            for snd in sends:
                snd.wait_send()

        for dev, children in TREE_CHILDREN.items():
            if dev == 0:
                continue

            @pl.when(my == dev)
            def _(children=children):
                sends = []
                for c in range(N_CHUNK):
                    chunk_rdma(c, 0, 0).wait_recv()
                    for j, child in enumerate(children):
                        snd = chunk_rdma(c, j, child)
                        snd.start()
                        sends.append(snd)
                for snd in sends:
                    snd.wait_send()

        leaves = [d for d in range(1, N_DEV) if d not in TREE_CHILDREN]

        @pl.when(sum((my == d) for d in leaves) > 0)
        def _():
            for c in range(N_CHUNK):
                chunk_rdma(c, 0, 0).wait_recv()

        return

    return pl.pallas_call(
        body,
        out_shape=jax.ShapeDtypeStruct((1, SQ, D), jnp.bfloat16),
        in_specs=[
            pl.BlockSpec(memory_space=pltpu.VMEM),
            pl.BlockSpec(memory_space=pltpu.VMEM),
            pl.BlockSpec(memory_space=pltpu.MemorySpace.HBM),
            pl.BlockSpec(memory_space=pltpu.MemorySpace.HBM),
            pl.BlockSpec(memory_space=pltpu.VMEM),
        ],
        out_specs=pl.BlockSpec(memory_space=pltpu.VMEM),
        scratch_shapes=[
            pltpu.VMEM((KTOT, D), jnp.bfloat16),
            pltpu.VMEM((KTOT, D), jnp.bfloat16),
            pltpu.VMEM((2, SQ, HQ, DH), jnp.float32),
            pltpu.VMEM((2, EDGE, HQ, DH), jnp.float32),
            pltpu.VMEM((2, EDGE, D), jnp.bfloat16),
            pltpu.SemaphoreType.DMA((2,)),
            pltpu.SemaphoreType.DMA,
            pltpu.SemaphoreType.DMA,
            pltpu.SemaphoreType.DMA((N_CHUNK, MAX_FANOUT)),
            pltpu.SemaphoreType.DMA((N_CHUNK,)),
        ],
        compiler_params=pltpu.CompilerParams(
            vmem_limit_bytes=100 * 1024 * 1024,
        ),
    )(x, Wq, K_ext, V_ext, Wo)
